# baseline (speedup 1.0000x reference)
import functools

import jax
import jax.numpy as jnp
from jax import lax
from jax.experimental import pallas as pl
from jax.experimental.pallas import tpu as pltpu
from jax.experimental.pallas import tpu_sc as plsc

_VOCAB = 1000000
_EMBED = 32
_BATCH = 16384
_CTX = 20

_NUM_TILES = 32

_A_CHUNK = 256
_A_MAIN = 999424
_A_CHUNKS_PER_TILE = _A_MAIN // (_A_CHUNK * _NUM_TILES)
_A_PER_TILE = _A_MAIN // _NUM_TILES
_A_TAIL = _VOCAB - _A_MAIN

_B_PER_TILE = _BATCH // _NUM_TILES
_GATHER_W = 128


def _sc_project_main(table, wb):
  mesh = plsc.VectorSubcoreMesh(core_axis_name="c", subcore_axis_name="s")

  @functools.partial(
      pl.kernel,
      out_type=jax.ShapeDtypeStruct((_A_MAIN,), jnp.float32),
      mesh=mesh,
      compiler_params=pltpu.CompilerParams(
          use_tc_tiling_on_sc=True, needs_layout_passes=False),
      scratch_types=[
          pltpu.VMEM((_A_CHUNK, _EMBED), jnp.float32),
          pltpu.VMEM((_A_CHUNK, _EMBED), jnp.float32),
          pltpu.VMEM((_EMBED, 16), jnp.float32),
          pltpu.VMEM((_A_PER_TILE,), jnp.float32),
          pltpu.SemaphoreType.DMA,
          pltpu.SemaphoreType.DMA,
      ],
  )
  def proj_kernel(table_hbm, wb_hbm, tw_hbm, buf0, buf1, wv, outv, sem,
                  sem_out):
    wid = lax.axis_index("s") * 2 + lax.axis_index("c")
    row0 = wid * _A_PER_TILE
    pltpu.sync_copy(wb_hbm, wv)

    wregs = [wv[l, pl.ds(0, 16)] for l in range(_EMBED)]
    iota = lax.iota(jnp.int32, 16)

    def start(buf, t):
      pltpu.async_copy(
          table_hbm.at[pl.ds(row0 + t * _A_CHUNK, _A_CHUNK), :], buf, sem
      )

    def wait(buf):
      pltpu.make_async_copy(
          table_hbm.at[pl.ds(0, _A_CHUNK), :], buf, sem
      ).wait()

    def compute(buf, t):
      @pl.loop(0, _A_CHUNK // 16)
      def _(g):
        row16 = g * 16 + iota
        accs = [None, None, None, None]
        for l in range(_EMBED):
          lane16 = jnp.full((16,), l, jnp.int32)
          v = plsc.load_gather(buf, [row16, lane16]) * wregs[l]
          k = l % 4
          accs[k] = v if accs[k] is None else accs[k] + v
        s = (accs[0] + accs[1]) + (accs[2] + accs[3])
        outv[pl.ds(t * _A_CHUNK + g * 16, 16)] = s

    start(buf0, 0)
    start(buf1, 1)

    @pl.loop(0, _A_CHUNKS_PER_TILE // 2)
    def _(T):
      t0 = T * 2
      wait(buf0)
      compute(buf0, t0)

      @pl.when(T < _A_CHUNKS_PER_TILE // 2 - 1)
      def _():
        start(buf0, t0 + 2)

      wait(buf1)
      compute(buf1, t0 + 1)

      @pl.when(T < _A_CHUNKS_PER_TILE // 2 - 1)
      def _():
        start(buf1, t0 + 3)

    pltpu.async_copy(outv, tw_hbm.at[pl.ds(row0, _A_PER_TILE)], sem_out)
    pltpu.make_async_copy(
        tw_hbm.at[pl.ds(0, _A_PER_TILE)], outv, sem_out
    ).wait()

  return proj_kernel(table, wb)


def _tc_project_tail(table_tail, W):
  def body(x_ref, w_ref, out_ref):
    prod = x_ref[...] * w_ref[...]
    out_ref[...] = jnp.sum(prod, axis=1, keepdims=True)

  return pl.pallas_call(
      body,
      out_shape=jax.ShapeDtypeStruct((_A_TAIL, 1), jnp.float32),
      in_specs=[
          pl.BlockSpec(memory_space=pltpu.VMEM),
          pl.BlockSpec(memory_space=pltpu.VMEM),
      ],
      out_specs=pl.BlockSpec(memory_space=pltpu.VMEM),
  )(table_tail, W)


def _sc_gather_sum(tw, inputs, bias16):
  mesh = plsc.VectorSubcoreMesh(core_axis_name="c", subcore_axis_name="s")
  n_groups = _B_PER_TILE // 16
  n_win = _B_PER_TILE // _GATHER_W

  @functools.partial(
      pl.kernel,
      out_type=jax.ShapeDtypeStruct((_BATCH,), jnp.float32),
      mesh=mesh,
      compiler_params=pltpu.CompilerParams(
          use_tc_tiling_on_sc=False, needs_layout_passes=False),
      scratch_types=[
          pltpu.VMEM((_B_PER_TILE, _CTX), jnp.int32),
          pltpu.VMEM((_CTX, _B_PER_TILE), jnp.int32),
          pltpu.VMEM((_CTX, _B_PER_TILE), jnp.float32),
          pltpu.VMEM((_B_PER_TILE,), jnp.float32),
          pltpu.VMEM((16,), jnp.float32),
          pltpu.SemaphoreType.DMA,
      ],
  )
  def gather_kernel(tw_hbm, idx_hbm, b_hbm, out_hbm, idx_v, idxT_v, vals_v,
                    out_v, b_v, sem):
    wid = lax.axis_index("s") * 2 + lax.axis_index("c")
    base = wid * _B_PER_TILE
    pltpu.sync_copy(idx_hbm.at[pl.ds(base, _B_PER_TILE), :], idx_v)
    pltpu.sync_copy(b_hbm, b_v)

    iota = lax.iota(jnp.int32, 16)

    @pl.loop(0, n_groups)
    def _(g):
      row16 = g * 16 + iota
      for j in range(_CTX):
        lane16 = jnp.full((16,), j, jnp.int32)
        idxT_v[j, pl.ds(g * 16, 16)] = plsc.load_gather(
            idx_v, [row16, lane16])

    @pl.loop(0, _CTX)
    def _(j):
      for k in range(n_win):
        pltpu.async_copy(
            tw_hbm.at[idxT_v.at[j, pl.ds(k * _GATHER_W, _GATHER_W)]],
            vals_v.at[j, pl.ds(k * _GATHER_W, _GATHER_W)],
            sem,
        )

    @pl.loop(0, _CTX * n_win)
    def _(d):
      pltpu.make_async_copy(
          tw_hbm.at[pl.ds(0, _GATHER_W)],
          vals_v.at[0, pl.ds(0, _GATHER_W)],
          sem,
      ).wait()

    inv = 1.0 / _CTX
    bvec = b_v[pl.ds(0, 16)]

    @pl.loop(0, n_groups)
    def _(g):
      sl = pl.ds(g * 16, 16)
      acc = vals_v[0, sl]
      for j in range(1, _CTX):
        acc += vals_v[j, sl]
      out_v[sl] = acc * inv + bvec

    pltpu.sync_copy(out_v, out_hbm.at[pl.ds(base, _B_PER_TILE)])

  return gather_kernel(tw, inputs, bias16)


@jax.jit
def kernel(inputs, table, W, b):
  wb = jnp.broadcast_to(W.reshape(_EMBED, 1), (_EMBED, 16))
  bias16 = jnp.broadcast_to(b, (16,))
  table_tail = jax.lax.slice(table, (_A_MAIN, 0), (_VOCAB, _EMBED))

  tw_main = _sc_project_main(table, wb)
  tw_tail = _tc_project_tail(table_tail, W).reshape(_A_TAIL)
  tw = jnp.concatenate([tw_main, tw_tail])

  out = _sc_gather_sum(tw, inputs, bias16)
  return out.reshape(_BATCH, 1)


# speedup vs baseline: 1.3183x; 1.3183x over previous
import functools

import jax
import jax.numpy as jnp
from jax import lax
from jax.experimental import pallas as pl
from jax.experimental.pallas import tpu as pltpu
from jax.experimental.pallas import tpu_sc as plsc

_VOCAB = 1000000
_EMBED = 32
_BATCH = 16384
_CTX = 20

_CHUNK_B = 64
_CHUNK_ROWS = _CHUNK_B * _CTX


def _sc_pool(table, inputs):
  mesh = plsc.VectorSubcoreMesh(core_axis_name="c", subcore_axis_name="s")
  n_chunks = _BATCH // _CHUNK_B

  @functools.partial(
      pl.kernel,
      out_type=jax.ShapeDtypeStruct((_BATCH, _EMBED), jnp.float32),
      mesh=mesh,
      compiler_params=pltpu.CompilerParams(use_tc_tiling_on_sc=False),
      scratch_types=[
          pltpu.VMEM((_CHUNK_ROWS, _EMBED), jnp.float32),
          pltpu.SemaphoreType.DMA,
      ],
  )
  def pool_kernel(table_hbm, idx_hbm, out_hbm, rows_v, sem):
    def body(idx_v, out_v):
      @pl.loop(0, _CHUNK_B)
      def _(b):
        pltpu.async_copy(
            table_hbm.at[idx_v.at[b]],
            rows_v.at[pl.ds(b * _CTX, _CTX)],
            sem,
        )
      pltpu.make_async_copy(
          table_hbm.at[pl.ds(0, _CHUNK_ROWS)], rows_v, sem
      ).wait()

      @pl.loop(0, _CHUNK_B)
      def _(b):
        base = b * _CTX
        s0 = rows_v[base, pl.ds(0, 16)]
        s1 = rows_v[base, pl.ds(16, 16)]
        for j in range(1, _CTX):
          s0 += rows_v[base + j, pl.ds(0, 16)]
          s1 += rows_v[base + j, pl.ds(16, 16)]
        out_v[b, pl.ds(0, 16)] = s0
        out_v[b, pl.ds(16, 16)] = s1

    pltpu.emit_pipeline(
        body,
        grid=(n_chunks,),
        in_specs=[
            pl.BlockSpec((_CHUNK_B, _CTX), index_map=lambda i: (i, 0)),
        ],
        out_specs=[
            pl.BlockSpec((_CHUNK_B, _EMBED), index_map=lambda i: (i, 0)),
        ],
        core_axis_name=("c", "s"),
        dimension_semantics=(pltpu.PARALLEL,),
    )(idx_hbm, out_hbm)

  return pool_kernel(table, inputs)


def _tc_project(pooled, W, b):
  def proj_kernel(pooled_ref, w_ref, b_ref, out_ref):
    w_row = w_ref[...]
    prod = pooled_ref[...] * w_row
    s = jnp.sum(prod, axis=1, keepdims=True)
    out_ref[...] = s * (1.0 / _CTX) + b_ref[0, 0]

  return pl.pallas_call(
      proj_kernel,
      out_shape=jax.ShapeDtypeStruct((_BATCH, 1), jnp.float32),
      in_specs=[
          pl.BlockSpec(memory_space=pltpu.VMEM),
          pl.BlockSpec(memory_space=pltpu.VMEM),
          pl.BlockSpec(memory_space=pltpu.SMEM),
      ],
      out_specs=pl.BlockSpec(memory_space=pltpu.VMEM),
  )(pooled, W, b.reshape(1, 1))


@jax.jit
def kernel(inputs, table, W, b):
  pooled = _sc_pool(table, inputs)
  return _tc_project(pooled, W, b)
